# baseline (speedup 1.0000x reference)
import functools

import jax
import jax.numpy as jnp
from jax.experimental import pallas as pl
from jax.experimental.pallas import tpu as pltpu

CONF_THRES = 0.25
IOU_THRES = 0.45
TOPK = 1000
PAD = 1024
MAX_DET = 300
ODET = 304
MAX_WH = 4096.0


def _score_body(p_ref, s_ref):
    p = p_ref[...]
    obj = jax.nn.sigmoid(p[:, 4:5])
    conf = obj * jax.nn.sigmoid(p[:, 5:85])
    s_ref[...] = jnp.max(conf, axis=1, keepdims=True)


def _scores(pred2d):
    n = pred2d.shape[0]
    blk = 4000
    return pl.pallas_call(
        _score_body,
        grid=(n // blk,),
        in_specs=[pl.BlockSpec((blk, 85), lambda i: (i, 0))],
        out_specs=pl.BlockSpec((blk, 1), lambda i: (i, 0)),
        out_shape=jax.ShapeDtypeStruct((n, 1), jnp.float32),
    )(pred2d)


def _nms_body(c_ref, o_ref, bmat, fmat):
    c = c_ref[0]

    xy = jax.nn.sigmoid(c[:, 0:2]) * 640.0
    wh = jax.nn.sigmoid(c[:, 2:4]) * 128.0
    x1y1 = xy - wh * 0.5
    x2y2 = xy + wh * 0.5
    conf = jax.nn.sigmoid(c[:, 4:5]) * jax.nn.sigmoid(c[:, 5:85])
    score = jnp.max(conf, axis=1, keepdims=True)
    ii = jax.lax.broadcasted_iota(jnp.int32, (PAD, 80), 1)
    cls = jnp.min(jnp.where(conf == score, ii, 127), axis=1, keepdims=True)
    clsf = cls.astype(jnp.float32)
    off = clsf * MAX_WH
    ox1 = x1y1[:, 0:1] + off
    oy1 = x1y1[:, 1:2] + off
    ox2 = x2y2[:, 0:1] + off
    oy2 = x2y2[:, 1:2] + off
    area = (ox2 - ox1) * (oy2 - oy1)

    g = jnp.concatenate([ox1, oy1, ox2, oy2, area, score,
                         jnp.zeros((PAD, 2), jnp.float32)], axis=1)
    eye8 = jnp.where(
        jax.lax.broadcasted_iota(jnp.int32, (8, 8), 0)
        == jax.lax.broadcasted_iota(jnp.int32, (8, 8), 1), 1.0, 0.0)
    gT = jax.lax.dot_general(eye8, g, (((1,), (1,)), ((), ())),
                             precision=jax.lax.Precision.HIGHEST,
                             preferred_element_type=jnp.float32)
    ox1T = gT[0:1, :]
    oy1T = gT[1:2, :]
    ox2T = gT[2:3, :]
    oy2T = gT[3:4, :]
    areaT = gT[4:5, :]
    scoreT = gT[5:6, :]

    for k in range(PAD // 128):
        s = slice(k * 128, (k + 1) * 128)
        ltx = jnp.maximum(ox1[s, :], ox1T)
        lty = jnp.maximum(oy1[s, :], oy1T)
        rbx = jnp.minimum(ox2[s, :], ox2T)
        rby = jnp.minimum(oy2[s, :], oy2T)
        w = jnp.clip(rbx - ltx, 0.0, None)
        h = jnp.clip(rby - lty, 0.0, None)
        inter = w * h
        iou = inter / (area[s, :] + areaT - inter + 1e-9)
        ri = jax.lax.broadcasted_iota(jnp.int32, (128, PAD), 0) + k * 128
        ci = jax.lax.broadcasted_iota(jnp.int32, (128, PAD), 1)
        bmat[s, :] = jnp.where((iou > IOU_THRES) & (ci > ri), 1.0, 0.0)

    lane = jax.lax.broadcasted_iota(jnp.int32, (1, PAD), 1)
    keep0 = jnp.where((scoreT > CONF_THRES) & (lane < TOPK), 1.0, 0.0)

    def body(i, keep):
        row = bmat[pl.ds(i, 1), :]
        keep_i = jnp.max(jnp.where(lane == i, keep, 0.0))
        return keep * (1.0 - row * keep_i)

    keep = jax.lax.fori_loop(0, TOPK, body, keep0)

    for k in range(PAD // 128):
        s = slice(k * 128, (k + 1) * 128)
        ri = jax.lax.broadcasted_iota(jnp.int32, (128, PAD), 0) + k * 128
        ci = jax.lax.broadcasted_iota(jnp.int32, (128, PAD), 1)
        bmat[s, :] = jnp.where(ri <= ci, 1.0, 0.0)
    keep8 = jnp.broadcast_to(keep, (8, PAD))
    ksum = jnp.dot(keep8, bmat[...], precision=jax.lax.Precision.HIGHEST,
                   preferred_element_type=jnp.float32)[0:1, :]

    fmat[:, 0:2] = x1y1
    fmat[:, 2:4] = x2y2
    fmat[:, 4:5] = score
    fmat[:, 5:6] = clsf
    fmat[:, 6:8] = jnp.zeros((PAD, 2), jnp.float32)
    oidx = (jax.lax.broadcasted_iota(jnp.int32, (ODET, PAD), 0) + 1
            ).astype(jnp.float32)
    sel = jnp.where((jnp.broadcast_to(ksum, (ODET, PAD)) == oidx)
                    & (jnp.broadcast_to(keep, (ODET, PAD)) > 0.0), 1.0, 0.0)
    o_ref[0] = jnp.dot(sel, fmat[...], precision=jax.lax.Precision.HIGHEST,
                       preferred_element_type=jnp.float32)


def _nms(cand):
    b = cand.shape[0]
    return pl.pallas_call(
        _nms_body,
        grid=(b,),
        in_specs=[
            pl.BlockSpec((1, PAD, 85), lambda i: (i, 0, 0)),
        ],
        out_specs=pl.BlockSpec((1, ODET, 8), lambda i: (i, 0, 0)),
        out_shape=jax.ShapeDtypeStruct((b, ODET, 8), jnp.float32),
        scratch_shapes=[
            pltpu.VMEM((PAD, PAD), jnp.float32),
            pltpu.VMEM((PAD, 8), jnp.float32),
        ],
    )(cand)


@jax.jit
def kernel(pred):
    b, n, f = pred.shape
    scores = _scores(pred.reshape(b * n, f)).reshape(b, n)
    _, idx = jax.lax.top_k(scores, TOPK)
    idx = jnp.pad(idx, ((0, 0), (0, PAD - TOPK)))
    cand = jnp.take_along_axis(pred, idx[:, :, None], axis=1)
    out = _nms(cand)
    return out[:, :MAX_DET, :6]


# speedup vs baseline: 1.0093x; 1.0093x over previous
import functools

import jax
import jax.numpy as jnp
from jax.experimental import pallas as pl
from jax.experimental.pallas import tpu as pltpu

CONF_THRES = 0.25
IOU_THRES = 0.45
TOPK = 1000
PAD = 1024
MAX_DET = 300
ODET = 304
MAX_WH = 4096.0


def _score_body(p_ref, s_ref):
    p = p_ref[...]
    obj = jax.nn.sigmoid(p[:, 4:5])
    conf = obj * jax.nn.sigmoid(p[:, 5:85])
    s_ref[...] = jnp.max(conf, axis=1, keepdims=True)


def _scores(pred2d):
    n = pred2d.shape[0]
    blk = 4000
    return pl.pallas_call(
        _score_body,
        grid=(n // blk,),
        in_specs=[pl.BlockSpec((blk, 85), lambda i: (i, 0))],
        out_specs=pl.BlockSpec((blk, 1), lambda i: (i, 0)),
        out_shape=jax.ShapeDtypeStruct((n, 1), jnp.float32),
    )(pred2d)


def _nms_body(c_ref, o_ref, bmat, dblk, fmat):
    c = c_ref[0]

    xy = jax.nn.sigmoid(c[:, 0:2]) * 640.0
    wh = jax.nn.sigmoid(c[:, 2:4]) * 128.0
    x1y1 = xy - wh * 0.5
    x2y2 = xy + wh * 0.5
    conf = jax.nn.sigmoid(c[:, 4:5]) * jax.nn.sigmoid(c[:, 5:85])
    score = jnp.max(conf, axis=1, keepdims=True)
    ii = jax.lax.broadcasted_iota(jnp.int32, (PAD, 80), 1)
    cls = jnp.min(jnp.where(conf == score, ii, 127), axis=1, keepdims=True)
    clsf = cls.astype(jnp.float32)
    off = clsf * MAX_WH
    ox1 = x1y1[:, 0:1] + off
    oy1 = x1y1[:, 1:2] + off
    ox2 = x2y2[:, 0:1] + off
    oy2 = x2y2[:, 1:2] + off
    area = (ox2 - ox1) * (oy2 - oy1)

    g = jnp.concatenate([ox1, oy1, ox2, oy2, area, score,
                         jnp.zeros((PAD, 2), jnp.float32)], axis=1)
    eye8 = jnp.where(
        jax.lax.broadcasted_iota(jnp.int32, (8, 8), 0)
        == jax.lax.broadcasted_iota(jnp.int32, (8, 8), 1), 1.0, 0.0)
    gT = jax.lax.dot_general(eye8, g, (((1,), (1,)), ((), ())),
                             precision=jax.lax.Precision.HIGHEST,
                             preferred_element_type=jnp.float32)
    ox1T = gT[0:1, :]
    oy1T = gT[1:2, :]
    ox2T = gT[2:3, :]
    oy2T = gT[3:4, :]
    areaT = gT[4:5, :]
    scoreT = gT[5:6, :]

    for k in range(PAD // 128):
        s = slice(k * 128, (k + 1) * 128)
        ltx = jnp.maximum(ox1[s, :], ox1T)
        lty = jnp.maximum(oy1[s, :], oy1T)
        rbx = jnp.minimum(ox2[s, :], ox2T)
        rby = jnp.minimum(oy2[s, :], oy2T)
        w = jnp.clip(rbx - ltx, 0.0, None)
        h = jnp.clip(rby - lty, 0.0, None)
        inter = w * h
        iou = inter / (area[s, :] + areaT - inter + 1e-9)
        ri = jax.lax.broadcasted_iota(jnp.int32, (128, PAD), 0) + k * 128
        ci = jax.lax.broadcasted_iota(jnp.int32, (128, PAD), 1)
        blk = jnp.where((iou > IOU_THRES) & (ci > ri), 1.0, 0.0)
        bmat[s, :] = blk
        dblk[s, :] = blk[:, k * 128:(k + 1) * 128]

    lane = jax.lax.broadcasted_iota(jnp.int32, (1, PAD), 1)
    keep = jnp.where((scoreT > CONF_THRES) & (lane < TOPK), 1.0, 0.0)
    lane128 = jax.lax.broadcasted_iota(jnp.int32, (1, 128), 1)
    nblk = PAD // 128
    segs = []
    rest = keep
    for k in range(nblk):
        kb = k * 128
        kblk = rest[:, 0:128]

        def body(i, kb_):
            row = dblk[pl.ds(kb + i, 1), :]
            keep_i = jnp.max(jnp.where(lane128 == i, kb_, 0.0))
            return kb_ * (1.0 - row * keep_i)

        kblk = jax.lax.fori_loop(0, 128, body, kblk)
        segs.append(kblk)
        if k + 1 < nblk:
            kb8 = jnp.broadcast_to(kblk, (8, 128))
            supp = jax.lax.dot_general(
                kb8, bmat[kb:kb + 128, kb + 128:],
                (((1,), (0,)), ((), ())),
                precision=jax.lax.Precision.HIGHEST,
                preferred_element_type=jnp.float32)[0:1, :]
            rest = rest[:, 128:] * jnp.where(supp > 0.0, 0.0, 1.0)
    keep = jnp.concatenate(segs, axis=1)

    for k in range(PAD // 128):
        s = slice(k * 128, (k + 1) * 128)
        ri = jax.lax.broadcasted_iota(jnp.int32, (128, PAD), 0) + k * 128
        ci = jax.lax.broadcasted_iota(jnp.int32, (128, PAD), 1)
        bmat[s, :] = jnp.where(ri <= ci, 1.0, 0.0)
    keep8 = jnp.broadcast_to(keep, (8, PAD))
    ksum = jnp.dot(keep8, bmat[...], precision=jax.lax.Precision.HIGHEST,
                   preferred_element_type=jnp.float32)[0:1, :]

    fmat[:, 0:2] = x1y1
    fmat[:, 2:4] = x2y2
    fmat[:, 4:5] = score
    fmat[:, 5:6] = clsf
    fmat[:, 6:8] = jnp.zeros((PAD, 2), jnp.float32)
    oidx = (jax.lax.broadcasted_iota(jnp.int32, (ODET, PAD), 0) + 1
            ).astype(jnp.float32)
    sel = jnp.where((jnp.broadcast_to(ksum, (ODET, PAD)) == oidx)
                    & (jnp.broadcast_to(keep, (ODET, PAD)) > 0.0), 1.0, 0.0)
    o_ref[0] = jnp.dot(sel, fmat[...], precision=jax.lax.Precision.HIGHEST,
                       preferred_element_type=jnp.float32)


def _nms(cand):
    b = cand.shape[0]
    return pl.pallas_call(
        _nms_body,
        grid=(b,),
        in_specs=[
            pl.BlockSpec((1, PAD, 85), lambda i: (i, 0, 0)),
        ],
        out_specs=pl.BlockSpec((1, ODET, 8), lambda i: (i, 0, 0)),
        out_shape=jax.ShapeDtypeStruct((b, ODET, 8), jnp.float32),
        scratch_shapes=[
            pltpu.VMEM((PAD, PAD), jnp.float32),
            pltpu.VMEM((PAD, 128), jnp.float32),
            pltpu.VMEM((PAD, 8), jnp.float32),
        ],
    )(cand)


@jax.jit
def kernel(pred):
    b, n, f = pred.shape
    scores = _scores(pred.reshape(b * n, f)).reshape(b, n)
    _, idx = jax.lax.top_k(scores, TOPK)
    idx = jnp.pad(idx, ((0, 0), (0, PAD - TOPK)))
    cand = jnp.take_along_axis(pred, idx[:, :, None], axis=1)
    out = _nms(cand)
    return out[:, :MAX_DET, :6]


# speedup vs baseline: 1.0171x; 1.0077x over previous
import functools

import jax
import jax.numpy as jnp
from jax.experimental import pallas as pl
from jax.experimental.pallas import tpu as pltpu

CONF_THRES = 0.25
IOU_THRES = 0.45
TOPK = 1000
PAD = 1024
MAX_DET = 300
ODET = 304
MAX_WH = 4096.0


def _score_body(p_ref, s_ref):
    p = p_ref[...]
    obj = jax.nn.sigmoid(p[:, 4:5])
    conf = obj * jax.nn.sigmoid(p[:, 5:85])
    s_ref[...] = jnp.max(conf, axis=1, keepdims=True)


def _scores(pred2d):
    n = pred2d.shape[0]
    blk = 4000
    return pl.pallas_call(
        _score_body,
        grid=(n // blk,),
        in_specs=[pl.BlockSpec((blk, 85), lambda i: (i, 0))],
        out_specs=pl.BlockSpec((blk, 1), lambda i: (i, 0)),
        out_shape=jax.ShapeDtypeStruct((n, 1), jnp.float32),
    )(pred2d)


def _nms_body(c_ref, o_ref, bmat, dblk, fmat):
    c = c_ref[0]

    xy = jax.nn.sigmoid(c[:, 0:2]) * 640.0
    wh = jax.nn.sigmoid(c[:, 2:4]) * 128.0
    x1y1 = xy - wh * 0.5
    x2y2 = xy + wh * 0.5
    conf = jax.nn.sigmoid(c[:, 4:5]) * jax.nn.sigmoid(c[:, 5:85])
    score = jnp.max(conf, axis=1, keepdims=True)
    ii = jax.lax.broadcasted_iota(jnp.int32, (PAD, 80), 1)
    cls = jnp.min(jnp.where(conf == score, ii, 127), axis=1, keepdims=True)
    clsf = cls.astype(jnp.float32)
    off = clsf * MAX_WH
    ox1 = x1y1[:, 0:1] + off
    oy1 = x1y1[:, 1:2] + off
    ox2 = x2y2[:, 0:1] + off
    oy2 = x2y2[:, 1:2] + off
    area = (ox2 - ox1) * (oy2 - oy1)

    g = jnp.concatenate([ox1, oy1, ox2, oy2, area, score,
                         jnp.zeros((PAD, 2), jnp.float32)], axis=1)
    eye8 = jnp.where(
        jax.lax.broadcasted_iota(jnp.int32, (8, 8), 0)
        == jax.lax.broadcasted_iota(jnp.int32, (8, 8), 1), 1.0, 0.0)
    gT = jax.lax.dot_general(eye8, g, (((1,), (1,)), ((), ())),
                             precision=jax.lax.Precision.HIGHEST,
                             preferred_element_type=jnp.float32)
    ox1T = gT[0:1, :]
    oy1T = gT[1:2, :]
    ox2T = gT[2:3, :]
    oy2T = gT[3:4, :]
    areaT = gT[4:5, :]
    scoreT = gT[5:6, :]

    for k in range(PAD // 128):
        s = slice(k * 128, (k + 1) * 128)
        ltx = jnp.maximum(ox1[s, :], ox1T)
        lty = jnp.maximum(oy1[s, :], oy1T)
        rbx = jnp.minimum(ox2[s, :], ox2T)
        rby = jnp.minimum(oy2[s, :], oy2T)
        w = jnp.clip(rbx - ltx, 0.0, None)
        h = jnp.clip(rby - lty, 0.0, None)
        inter = w * h
        iou = inter / (area[s, :] + areaT - inter + 1e-9)
        ri = jax.lax.broadcasted_iota(jnp.int32, (128, PAD), 0) + k * 128
        ci = jax.lax.broadcasted_iota(jnp.int32, (128, PAD), 1)
        blk = jnp.where((iou > IOU_THRES) & (ci > ri), 1.0, 0.0)
        bmat[s, :] = blk
        dblk[s, :] = blk[:, k * 128:(k + 1) * 128]

    lane = jax.lax.broadcasted_iota(jnp.int32, (1, PAD), 1)
    keep = jnp.where((scoreT > CONF_THRES) & (lane < TOPK), 1.0, 0.0)
    lane128 = jax.lax.broadcasted_iota(jnp.int32, (1, 128), 1)
    nblk = PAD // 128
    segs = []
    rest = keep
    for k in range(nblk):
        kb = k * 128
        kblk = rest[:, 0:128]

        def body(t, kb_):
            r8 = dblk[pl.ds(kb + t * 8, 8), :]
            for j in range(8):
                i = t * 8 + j
                keep_i = jnp.max(jnp.where(lane128 == i, kb_, 0.0))
                kb_ = kb_ * (1.0 - r8[j:j + 1, :] * keep_i)
            return kb_

        kblk = jax.lax.fori_loop(0, 16, body, kblk)
        segs.append(kblk)
        if k + 1 < nblk:
            kb8 = jnp.broadcast_to(kblk, (8, 128))
            supp = jax.lax.dot_general(
                kb8, bmat[kb:kb + 128, kb + 128:],
                (((1,), (0,)), ((), ())),
                precision=jax.lax.Precision.HIGHEST,
                preferred_element_type=jnp.float32)[0:1, :]
            rest = rest[:, 128:] * jnp.where(supp > 0.0, 0.0, 1.0)
    keep = jnp.concatenate(segs, axis=1)

    for k in range(PAD // 128):
        s = slice(k * 128, (k + 1) * 128)
        ri = jax.lax.broadcasted_iota(jnp.int32, (128, PAD), 0) + k * 128
        ci = jax.lax.broadcasted_iota(jnp.int32, (128, PAD), 1)
        bmat[s, :] = jnp.where(ri <= ci, 1.0, 0.0)
    keep8 = jnp.broadcast_to(keep, (8, PAD))
    ksum = jnp.dot(keep8, bmat[...], precision=jax.lax.Precision.HIGHEST,
                   preferred_element_type=jnp.float32)[0:1, :]

    fmat[:, 0:2] = x1y1
    fmat[:, 2:4] = x2y2
    fmat[:, 4:5] = score
    fmat[:, 5:6] = clsf
    fmat[:, 6:8] = jnp.zeros((PAD, 2), jnp.float32)
    oidx = (jax.lax.broadcasted_iota(jnp.int32, (ODET, PAD), 0) + 1
            ).astype(jnp.float32)
    sel = jnp.where((jnp.broadcast_to(ksum, (ODET, PAD)) == oidx)
                    & (jnp.broadcast_to(keep, (ODET, PAD)) > 0.0), 1.0, 0.0)
    o_ref[0] = jnp.dot(sel, fmat[...], precision=jax.lax.Precision.HIGHEST,
                       preferred_element_type=jnp.float32)


def _nms(cand):
    b = cand.shape[0]
    return pl.pallas_call(
        _nms_body,
        grid=(b,),
        in_specs=[
            pl.BlockSpec((1, PAD, 85), lambda i: (i, 0, 0)),
        ],
        out_specs=pl.BlockSpec((1, ODET, 8), lambda i: (i, 0, 0)),
        out_shape=jax.ShapeDtypeStruct((b, ODET, 8), jnp.float32),
        scratch_shapes=[
            pltpu.VMEM((PAD, PAD), jnp.float32),
            pltpu.VMEM((PAD, 128), jnp.float32),
            pltpu.VMEM((PAD, 8), jnp.float32),
        ],
    )(cand)


@jax.jit
def kernel(pred):
    b, n, f = pred.shape
    scores = _scores(pred.reshape(b * n, f)).reshape(b, n)
    _, idx = jax.lax.top_k(scores, TOPK)
    idx = jnp.pad(idx, ((0, 0), (0, PAD - TOPK)))
    cand = jnp.take_along_axis(pred, idx[:, :, None], axis=1)
    out = _nms(cand)
    return out[:, :MAX_DET, :6]
